# baseline (speedup 1.0000x reference)
import functools

import jax
import jax.numpy as jnp
from jax import lax
from jax.experimental import pallas as pl
from jax.experimental.pallas import tpu as pltpu
from jax.experimental.pallas import tpu_sc as plsc

N = 10000
N_PAD = 10240
E = 320000
NSUB = 16
NCORE = 2
EBLK = 128
NBLK = 160
E_PAD = NBLK * EBLK * NSUB
CH = 128
ROWS_PER_SUB = N_PAD // NSUB
NB_TC = 1024
GRID_TC = N_PAD // NB_TC


def _extract_lane(vec, lane):
    ids = lax.broadcasted_iota(jnp.int32, (16,), 0)
    return jnp.sum(jnp.where(ids == lane, vec, 0))


@functools.cache
def _make_segsum(nq):
    mesh = plsc.VectorSubcoreMesh(core_axis_name="c", subcore_axis_name="s")
    acc_rows = N_PAD // 2 if nq == 2 else N_PAD // 4
    sub_rng = acc_rows // NSUB
    S = 4

    def body(q3, sd2, blo, bhi, zeros_hbm, out3,
             blo_v, bhi_v, sdg, dstloc, rows_v, acc, sem_g, sem_s, sem_i):
        cid = lax.axis_index("c")
        sid = lax.axis_index("s")
        table = q3.at[cid] if nq == 2 else q3.at[0]
        out = out3.at[cid] if nq == 2 else out3.at[0]
        pltpu.sync_copy(blo, blo_v)
        pltpu.sync_copy(bhi, bhi_v)

        def run_pass(p):
            if nq == 2:
                pbase = p * acc_rows
            else:
                pbase = (p * NCORE + cid) * acc_rows
            pltpu.sync_copy(zeros_hbm.at[pl.ds(0, sub_rng)],
                            acc.at[pl.ds(sid * sub_rng, sub_rng)])
            lo = _extract_lane(blo_v[p, cid], sid)
            hi = _extract_lane(bhi_v[p, cid], sid)
            nit = hi - lo
            vlo = sid * sub_rng

            def scatter_t(t):
                return pltpu.make_async_copy(
                    rows_v.at[t], acc.at[dstloc.at[t]], sem_s.at[t])

            def idx_fetch(s, par):
                return pltpu.make_async_copy(sd2.at[s], sdg.at[par],
                                             sem_i.at[par])

            def process(par, first=None):
                for t in range(S):
                    if first is None:
                        scatter_t(t).wait()
                    else:
                        @pl.when(jnp.logical_not(first))
                        def _():
                            scatter_t(t).wait()

                    pltpu.async_copy(table.at[sdg.at[par].at[t]],
                                     rows_v.at[t], sem_g.at[t])
                for t in range(S):
                    pltpu.make_async_copy(table.at[sdg.at[par].at[t]],
                                          rows_v.at[t], sem_g.at[t]).wait()
                    for i8 in range(EBLK // 16):
                        d = sdg[par, S + t, pl.ds(16 * i8, 16)] - pbase
                        ok = jnp.logical_and(d >= vlo, d < vlo + sub_rng)
                        dstloc[t, pl.ds(16 * i8, 16)] = jnp.where(
                            ok, d, acc_rows)
                    pltpu.async_copy(rows_v.at[t], acc.at[dstloc.at[t]],
                                     sem_s.at[t], add=True)

            nit2 = (nit + 1) // 2

            @pl.when(nit2 > 0)
            def _():
                idx_fetch(lo, 0).start()

            def step(i, carry):
                sa = lo + 2 * i
                idx_fetch(sa, 0).wait()
                idx_fetch(sa + 1, 1).start()
                process(0, first=(i == 0))
                idx_fetch(sa + 1, 1).wait()
                idx_fetch(sa + 2, 0).start()
                process(1)
                return carry

            lax.fori_loop(0, nit2, step, 0)

            @pl.when(nit2 > 0)
            def _():
                idx_fetch(lo, 0).wait()
                for t in range(S):
                    scatter_t(t).wait()

            pltpu.sync_copy(acc.at[pl.ds(sid * sub_rng, sub_rng)],
                            out.at[pl.ds(pbase + sid * sub_rng, sub_rng)])

        for p in range(2):
            run_pass(p)

    return pl.kernel(
        body,
        out_type=jax.ShapeDtypeStruct((nq, N_PAD, CH), jnp.float32),
        mesh=mesh,
        compiler_params=pltpu.CompilerParams(needs_layout_passes=False),
        scratch_types=[
            pltpu.VMEM((2, NCORE, 16), jnp.int32),
            pltpu.VMEM((2, NCORE, 16), jnp.int32),
            pltpu.VMEM((2, 2 * S, EBLK), jnp.int32),
            pltpu.VMEM((S, EBLK), jnp.int32),
            pltpu.VMEM((S, EBLK, CH), jnp.float32),
            pltpu.VMEM_SHARED((acc_rows + 8, CH), jnp.float32),
            pltpu.SemaphoreType.DMA((S,)),
            pltpu.SemaphoreType.DMA((S,)),
            pltpu.SemaphoreType.DMA((2,)),
        ],
    )


@functools.cache
def _make_deg():
    mesh = plsc.VectorSubcoreMesh(core_axis_name="c", subcore_axis_name="s")
    nrow = N_PAD // EBLK
    blocks_per_w = (E_PAD // EBLK) // (NCORE * NSUB)

    def body(src2, zeros_hbm, out3, srcg, iden, hloc, acc, sem):
        cid = lax.axis_index("c")
        sid = lax.axis_index("s")
        w = cid * NSUB + sid
        pltpu.sync_copy(zeros_hbm.at[pl.ds(0, nrow)], hloc)
        for i in range(nrow // 16):
            iden[0, pl.ds(16 * i, 16)] = (
                lax.broadcasted_iota(jnp.int32, (16,), 0) + 16 * i)
        @pl.when(sid == 0)
        def _():
            pltpu.sync_copy(zeros_hbm.at[pl.ds(0, nrow)], acc)

        plsc.subcore_barrier()
        ones = jnp.full((16,), 1.0, jnp.float32)

        def step(j, carry):
            pltpu.sync_copy(src2.at[j], srcg)
            for i in range(EBLK // 16):
                v = srcg[pl.ds(16 * i, 16)]
                plsc.addupdate_scatter(
                    hloc, [lax.shift_right_logical(v, 7),
                           lax.bitwise_and(v, 127)], ones)
            return carry

        lax.fori_loop(w * blocks_per_w, (w + 1) * blocks_per_w, step, 0)
        pltpu.sync_copy(hloc, acc.at[iden.at[0]], add=True)
        plsc.subcore_barrier()

        @pl.when(sid == 0)
        def _():
            pltpu.sync_copy(acc, out3.at[cid])

    return pl.kernel(
        body,
        out_type=jax.ShapeDtypeStruct((NCORE, nrow, EBLK), jnp.float32),
        mesh=mesh,
        compiler_params=pltpu.CompilerParams(needs_layout_passes=False),
        scratch_types=[
            pltpu.VMEM((EBLK,), jnp.int32),
            pltpu.VMEM((1, nrow), jnp.int32),
            pltpu.VMEM((nrow, EBLK), jnp.float32),
            pltpu.VMEM_SHARED((nrow, EBLK), jnp.float32),
            pltpu.SemaphoreType.DMA,
        ],
    )


def _prep_body(deg_ref, dinv_ref):
    d = deg_ref[0] + deg_ref[1]
    nrow = N_PAD // EBLK
    node = (lax.broadcasted_iota(jnp.int32, (nrow, EBLK), 0) * EBLK
            + lax.broadcasted_iota(jnp.int32, (nrow, EBLK), 1))
    ok = jnp.logical_and(d > 0.0, node < N)
    dinv_ref[...] = jnp.where(ok, lax.rsqrt(jnp.maximum(d, 1e-30)), 0.0)


def _tc_prep(degs):
    return pl.pallas_call(
        _prep_body,
        out_shape=jax.ShapeDtypeStruct((N_PAD // EBLK, EBLK), jnp.float32),
    )(degs)


def _init_body_q(h_ref, dinv_ref, w_ref, q3_ref, acc_ref, *, nq):
    h = h_ref[...]
    q = h * dinv_ref[...]
    for j in range(nq):
        q3_ref[j] = q[:, j * CH:(j + 1) * CH]
    acc_ref[...] = jnp.dot(h, w_ref[...], preferred_element_type=jnp.float32)


def _init_body_noq(h_ref, dinv_ref, w_ref, acc_ref):
    acc_ref[...] = jnp.dot(h_ref[...], w_ref[...],
                           preferred_element_type=jnp.float32)


@functools.cache
def _make_init(c, hdim, with_q):
    nq = c // CH
    in_specs = [
        pl.BlockSpec((NB_TC, c), lambda i: (i, 0)),
        pl.BlockSpec((NB_TC, 1), lambda i: (i, 0)),
        pl.BlockSpec((c, hdim), lambda i: (0, 0)),
    ]
    acc_spec = pl.BlockSpec((NB_TC, hdim), lambda i: (i, 0))
    acc_shape = jax.ShapeDtypeStruct((N_PAD, hdim), jnp.float32)
    if with_q:
        return pl.pallas_call(
            functools.partial(_init_body_q, nq=nq),
            grid=(GRID_TC,),
            in_specs=in_specs,
            out_specs=[
                pl.BlockSpec((nq, NB_TC, CH), lambda i: (0, i, 0)),
                acc_spec,
            ],
            out_shape=[
                jax.ShapeDtypeStruct((nq, N_PAD, CH), jnp.float32),
                acc_shape,
            ],
        )
    return pl.pallas_call(
        _init_body_noq,
        grid=(GRID_TC,),
        in_specs=in_specs,
        out_specs=acc_spec,
        out_shape=acc_shape,
    )


def _step_body(a3_ref, tm2_ref, dinv_ref, w_ref, accin_ref,
               t_ref, q3_ref, accout_ref, *, nq, alpha, beta):
    if nq == 2:
        a = jnp.concatenate([a3_ref[0], a3_ref[1]], axis=1)
    else:
        a = a3_ref[0]
    dinv = dinv_ref[...]
    t = alpha * dinv * a
    if beta != 0.0:
        t = t + beta * tm2_ref[...]
    t_ref[...] = t
    q = dinv * t
    for j in range(nq):
        q3_ref[j] = q[:, j * CH:(j + 1) * CH]
    accout_ref[...] = accin_ref[...] + jnp.dot(
        t, w_ref[...], preferred_element_type=jnp.float32)


@functools.cache
def _make_step(c, hdim, alpha, beta):
    nq = c // CH
    kern = functools.partial(_step_body, nq=nq, alpha=alpha, beta=beta)
    out_shape = [
        jax.ShapeDtypeStruct((N_PAD, c), jnp.float32),
        jax.ShapeDtypeStruct((nq, N_PAD, CH), jnp.float32),
        jax.ShapeDtypeStruct((N_PAD, hdim), jnp.float32),
    ]
    return pl.pallas_call(
        kern,
        grid=(GRID_TC,),
        in_specs=[
            pl.BlockSpec((nq, NB_TC, CH), lambda i: (0, i, 0)),
            pl.BlockSpec((NB_TC, c), lambda i: (i, 0)),
            pl.BlockSpec((NB_TC, 1), lambda i: (i, 0)),
            pl.BlockSpec((c, hdim), lambda i: (0, 0)),
            pl.BlockSpec((NB_TC, hdim), lambda i: (i, 0)),
        ],
        out_specs=[
            pl.BlockSpec((NB_TC, c), lambda i: (i, 0)),
            pl.BlockSpec((nq, NB_TC, CH), lambda i: (0, i, 0)),
            pl.BlockSpec((NB_TC, hdim), lambda i: (i, 0)),
        ],
        out_shape=out_shape,
    )


def _stats_body(acc_ref, s_ref):
    i = pl.program_id(0)

    @pl.when(i == 0)
    def _():
        s_ref[...] = jnp.zeros_like(s_ref)

    x = acc_ref[...]
    s0 = jnp.sum(x, axis=0, keepdims=True)
    s1 = jnp.sum(x * x, axis=0, keepdims=True)
    s_ref[...] += jnp.concatenate([s0, s1], axis=0)


@functools.cache
def _make_stats(hdim):
    return pl.pallas_call(
        _stats_body,
        grid=(GRID_TC,),
        in_specs=[pl.BlockSpec((NB_TC, hdim), lambda i: (i, 0))],
        out_specs=pl.BlockSpec((2, hdim), lambda i: (0, 0)),
        out_shape=jax.ShapeDtypeStruct((2, hdim), jnp.float32),
    )


def _norm_body(acc_ref, s_ref, g_ref, be_ref, dinv_ref, h_ref, q3_ref, *, ch):
    i = pl.program_id(0)
    inv_n = 1.0 / N
    mean = s_ref[0:1, :] * inv_n
    var = s_ref[1:2, :] * inv_n - mean * mean
    y = g_ref[...] * (acc_ref[...] - mean) * lax.rsqrt(var + 1e-5) + be_ref[...]
    y = jnp.maximum(y, 0.0)
    rows = i * NB_TC + lax.broadcasted_iota(jnp.int32, y.shape, 0)
    y = jnp.where(rows < N, y, 0.0)
    h_ref[...] = y
    q = y * dinv_ref[...]
    q3_ref[0] = q[:, :ch]
    q3_ref[1] = q[:, ch:]


@functools.cache
def _make_norm(hdim):
    ch = hdim // 2
    kern = functools.partial(_norm_body, ch=ch)
    out_shape = [
        jax.ShapeDtypeStruct((N_PAD, hdim), jnp.float32),
        jax.ShapeDtypeStruct((NCORE, N_PAD, ch), jnp.float32),
    ]
    return pl.pallas_call(
        kern,
        grid=(GRID_TC,),
        in_specs=[
            pl.BlockSpec((NB_TC, hdim), lambda i: (i, 0)),
            pl.BlockSpec((2, hdim), lambda i: (0, 0)),
            pl.BlockSpec((1, hdim), lambda i: (0, 0)),
            pl.BlockSpec((1, hdim), lambda i: (0, 0)),
            pl.BlockSpec((NB_TC, 1), lambda i: (i, 0)),
        ],
        out_specs=[
            pl.BlockSpec((NB_TC, hdim), lambda i: (i, 0)),
            pl.BlockSpec((NCORE, NB_TC, ch), lambda i: (0, i, 0)),
        ],
        out_shape=out_shape,
    )


def _bias_body(acc_ref, b_ref, out_ref):
    out_ref[...] = acc_ref[...] + b_ref[...]


@functools.cache
def _make_bias(hdim):
    return pl.pallas_call(
        _bias_body,
        grid=(GRID_TC,),
        in_specs=[
            pl.BlockSpec((NB_TC, hdim), lambda i: (i, 0)),
            pl.BlockSpec((1, hdim), lambda i: (0, 0)),
        ],
        out_specs=pl.BlockSpec((NB_TC, hdim), lambda i: (i, 0)),
        out_shape=jax.ShapeDtypeStruct((N_PAD, hdim), jnp.float32),
    )


def _cheb_layer(q3, h, dinv, ed, W):
    K, c, hdim = W.shape
    nq = c // CH
    segsum = _make_segsum(nq)
    blo, bhi = (ed["b2lo"], ed["b2hi"]) if nq == 2 else (ed["b1lo"], ed["b1hi"])
    if q3 is None:
        q3, acc = _make_init(c, hdim, True)(h, dinv, W[0])
    else:
        acc = _make_init(c, hdim, False)(h, dinv, W[0])
    t_km1, t_km2 = h, h
    for k in range(1, K):
        a3 = segsum(q3, ed["sd"], blo, bhi, ed["zeros"])
        alpha, beta = (-1.0, 0.0) if k == 1 else (-2.0, -1.0)
        t, q3, acc = _make_step(c, hdim, alpha, beta)(a3, t_km2, dinv, W[k], acc)
        t_km2, t_km1 = t_km1, t
    return acc


def kernel(x, edge_index, W1, b1, g1, be1, W2, b2, g2, be2,
           W3, b3, g3, be3, W4, b4):
    x = x.astype(jnp.float32)
    src = edge_index[0].astype(jnp.int32)
    dst = edge_index[1].astype(jnp.int32)
    pad_e = E_PAD - E
    srcp = jnp.concatenate([src, jnp.full((pad_e,), N, jnp.int32)])
    dstp = jnp.concatenate([dst, jnp.full((pad_e,), N, jnp.int32)])

    order = jnp.argsort(dstp)
    dst_s = dstp[order]
    src_s = srcp[order]

    sb_edges = 4 * EBLK
    extra = 2 * sb_edges
    src_sb = jnp.concatenate(
        [src_s, jnp.full((extra,), N, jnp.int32)]).reshape(-1, 4, EBLK)
    dst_sb = jnp.concatenate(
        [dst_s, jnp.full((extra,), N, jnp.int32)]).reshape(-1, 4, EBLK)
    sd = jnp.concatenate([src_sb, dst_sb], axis=1)

    def sb_bounds(step):
        nodes = jnp.arange(0, N_PAD + 1, step, dtype=jnp.int32)
        e = jnp.searchsorted(dst_s, nodes, side="left").astype(jnp.int32)
        lo = e[:-1] // sb_edges
        hi = -((-e[1:]) // sb_edges)
        return lo, hi

    lo64, hi64 = sb_bounds(N_PAD // 64)
    lo32, hi32 = sb_bounds(N_PAD // 32)
    ed = {
        "sd": sd,
        "b1lo": lo64.reshape(2, 2, 16),
        "b1hi": hi64.reshape(2, 2, 16),
        "b2lo": jnp.broadcast_to(lo32.reshape(2, 1, 16), (2, 2, 16)),
        "b2hi": jnp.broadcast_to(hi32.reshape(2, 1, 16), (2, 2, 16)),
        "zeros": jnp.zeros((N_PAD // NSUB + 8, CH), jnp.float32),
    }

    x_pad = jnp.pad(x, ((0, N_PAD - N), (0, 0)))

    a_deg = _make_deg()(srcp.reshape(E_PAD // EBLK, EBLK), ed["zeros"])
    dinv = _tc_prep(a_deg).reshape(N_PAD, 1)

    h = x_pad
    q3 = None
    for (W, g, be) in ((W1, g1, be1), (W2, g2, be2), (W3, g3, be3)):
        acc = _cheb_layer(q3, h, dinv, ed, W)
        s = _make_stats(W.shape[2])(acc)
        h, q3 = _make_norm(W.shape[2])(acc, s, g.reshape(1, -1),
                                       be.reshape(1, -1), dinv)
    acc = _cheb_layer(q3, h, dinv, ed, W4)
    out = _make_bias(W4.shape[2])(acc, b4.reshape(1, -1))
    return out[:N]


# speedup vs baseline: 1.0093x; 1.0093x over previous
import functools

import jax
import jax.numpy as jnp
from jax import lax
from jax.experimental import pallas as pl
from jax.experimental.pallas import tpu as pltpu
from jax.experimental.pallas import tpu_sc as plsc

N = 10000
N_PAD = 10240
E = 320000
NSUB = 16
NCORE = 2
EBLK = 128
NBLK = 160
E_PAD = NBLK * EBLK * NSUB
CH = 128
SB_BLOCKS = 5
ROWS_PER_SUB = N_PAD // NSUB
NB_TC = 1024
GRID_TC = N_PAD // NB_TC


def _extract_lane(vec, lane):
    ids = lax.broadcasted_iota(jnp.int32, (16,), 0)
    return jnp.sum(jnp.where(ids == lane, vec, 0))


@functools.cache
def _make_segsum(nq):
    mesh = plsc.VectorSubcoreMesh(core_axis_name="c", subcore_axis_name="s")
    acc_rows = N_PAD // 2 if nq == 2 else N_PAD // 4
    sub_rng = acc_rows // NSUB
    S = SB_BLOCKS

    def body(q3, sd2, blo, bhi, zeros_hbm, out3,
             blo_v, bhi_v, sdg, dstloc, rows_v, acc, sem_g, sem_s):
        cid = lax.axis_index("c")
        sid = lax.axis_index("s")
        table = q3.at[cid] if nq == 2 else q3.at[0]
        out = out3.at[cid] if nq == 2 else out3.at[0]
        pltpu.sync_copy(blo, blo_v)
        pltpu.sync_copy(bhi, bhi_v)

        def run_pass(p):
            if nq == 2:
                pbase = p * acc_rows
            else:
                pbase = (p * NCORE + cid) * acc_rows
            pltpu.sync_copy(zeros_hbm.at[pl.ds(0, sub_rng)],
                            acc.at[pl.ds(sid * sub_rng, sub_rng)])
            lo = _extract_lane(blo_v[p, cid], sid)
            hi = _extract_lane(bhi_v[p, cid], sid)
            nit = hi - lo
            vlo = sid * sub_rng

            def scatter_t(t):
                return pltpu.make_async_copy(
                    rows_v.at[t], acc.at[dstloc.at[t]], sem_s.at[t])

            def step(i, carry):
                pltpu.sync_copy(sd2.at[lo + i], sdg)
                for t in range(S):
                    @pl.when(i > 0)
                    def _():
                        scatter_t(t).wait()

                    pltpu.async_copy(table.at[sdg.at[t]], rows_v.at[t],
                                     sem_g.at[t])
                for t in range(S):
                    for i8 in range(EBLK // 16):
                        d = sdg[S + t, pl.ds(16 * i8, 16)] - pbase
                        ok = jnp.logical_and(d >= vlo, d < vlo + sub_rng)
                        dstloc[t, pl.ds(16 * i8, 16)] = jnp.where(
                            ok, d, acc_rows)
                for t in range(S):
                    pltpu.make_async_copy(table.at[sdg.at[t]], rows_v.at[t],
                                          sem_g.at[t]).wait()
                    pltpu.async_copy(rows_v.at[t], acc.at[dstloc.at[t]],
                                     sem_s.at[t], add=True)
                return carry

            lax.fori_loop(0, nit, step, 0)

            @pl.when(nit > 0)
            def _():
                for t in range(S):
                    scatter_t(t).wait()

            pltpu.sync_copy(acc.at[pl.ds(sid * sub_rng, sub_rng)],
                            out.at[pl.ds(pbase + sid * sub_rng, sub_rng)])

        for p in range(2):
            run_pass(p)

    return pl.kernel(
        body,
        out_type=jax.ShapeDtypeStruct((nq, N_PAD, CH), jnp.float32),
        mesh=mesh,
        compiler_params=pltpu.CompilerParams(needs_layout_passes=False),
        scratch_types=[
            pltpu.VMEM((2, NCORE, 16), jnp.int32),
            pltpu.VMEM((2, NCORE, 16), jnp.int32),
            pltpu.VMEM((2 * S, EBLK), jnp.int32),
            pltpu.VMEM((S, EBLK), jnp.int32),
            pltpu.VMEM((S, EBLK, CH), jnp.float32),
            pltpu.VMEM_SHARED((acc_rows + 8, CH), jnp.float32),
            pltpu.SemaphoreType.DMA((S,)),
            pltpu.SemaphoreType.DMA((S,)),
        ],
    )


@functools.cache
def _make_deg():
    mesh = plsc.VectorSubcoreMesh(core_axis_name="c", subcore_axis_name="s")
    nrow = N_PAD // EBLK
    blocks_per_w = (E_PAD // EBLK) // (NCORE * NSUB)

    def body(src2, zeros_hbm, out3, srcg, iden, hloc, acc, sem):
        cid = lax.axis_index("c")
        sid = lax.axis_index("s")
        w = cid * NSUB + sid
        pltpu.sync_copy(zeros_hbm.at[pl.ds(0, nrow)], hloc)
        for i in range(nrow // 16):
            iden[0, pl.ds(16 * i, 16)] = (
                lax.broadcasted_iota(jnp.int32, (16,), 0) + 16 * i)
        @pl.when(sid == 0)
        def _():
            pltpu.sync_copy(zeros_hbm.at[pl.ds(0, nrow)], acc)

        plsc.subcore_barrier()
        ones = jnp.full((16,), 1.0, jnp.float32)

        def step(j, carry):
            pltpu.sync_copy(src2.at[j], srcg)
            for i in range(EBLK // 16):
                v = srcg[pl.ds(16 * i, 16)]
                plsc.addupdate_scatter(
                    hloc, [lax.shift_right_logical(v, 7),
                           lax.bitwise_and(v, 127)], ones)
            return carry

        lax.fori_loop(w * blocks_per_w, (w + 1) * blocks_per_w, step, 0)
        pltpu.sync_copy(hloc, acc.at[iden.at[0]], add=True)
        plsc.subcore_barrier()

        @pl.when(sid == 0)
        def _():
            pltpu.sync_copy(acc, out3.at[cid])

    return pl.kernel(
        body,
        out_type=jax.ShapeDtypeStruct((NCORE, nrow, EBLK), jnp.float32),
        mesh=mesh,
        compiler_params=pltpu.CompilerParams(needs_layout_passes=False),
        scratch_types=[
            pltpu.VMEM((EBLK,), jnp.int32),
            pltpu.VMEM((1, nrow), jnp.int32),
            pltpu.VMEM((nrow, EBLK), jnp.float32),
            pltpu.VMEM_SHARED((nrow, EBLK), jnp.float32),
            pltpu.SemaphoreType.DMA,
        ],
    )


def _prep_body(deg_ref, dinv_ref):
    d = deg_ref[0] + deg_ref[1]
    nrow = N_PAD // EBLK
    node = (lax.broadcasted_iota(jnp.int32, (nrow, EBLK), 0) * EBLK
            + lax.broadcasted_iota(jnp.int32, (nrow, EBLK), 1))
    ok = jnp.logical_and(d > 0.0, node < N)
    dinv_ref[...] = jnp.where(ok, lax.rsqrt(jnp.maximum(d, 1e-30)), 0.0)


def _tc_prep(degs):
    return pl.pallas_call(
        _prep_body,
        out_shape=jax.ShapeDtypeStruct((N_PAD // EBLK, EBLK), jnp.float32),
    )(degs)


def _init_body_q(h_ref, dinv_ref, w_ref, q3_ref, acc_ref, *, nq):
    h = h_ref[...]
    q = h * dinv_ref[...]
    for j in range(nq):
        q3_ref[j] = q[:, j * CH:(j + 1) * CH]
    acc_ref[...] = jnp.dot(h, w_ref[...], preferred_element_type=jnp.float32)


def _init_body_noq(h_ref, dinv_ref, w_ref, acc_ref):
    acc_ref[...] = jnp.dot(h_ref[...], w_ref[...],
                           preferred_element_type=jnp.float32)


@functools.cache
def _make_init(c, hdim, with_q):
    nq = c // CH
    in_specs = [
        pl.BlockSpec((NB_TC, c), lambda i: (i, 0)),
        pl.BlockSpec((NB_TC, 1), lambda i: (i, 0)),
        pl.BlockSpec((c, hdim), lambda i: (0, 0)),
    ]
    acc_spec = pl.BlockSpec((NB_TC, hdim), lambda i: (i, 0))
    acc_shape = jax.ShapeDtypeStruct((N_PAD, hdim), jnp.float32)
    if with_q:
        return pl.pallas_call(
            functools.partial(_init_body_q, nq=nq),
            grid=(GRID_TC,),
            in_specs=in_specs,
            out_specs=[
                pl.BlockSpec((nq, NB_TC, CH), lambda i: (0, i, 0)),
                acc_spec,
            ],
            out_shape=[
                jax.ShapeDtypeStruct((nq, N_PAD, CH), jnp.float32),
                acc_shape,
            ],
        )
    return pl.pallas_call(
        _init_body_noq,
        grid=(GRID_TC,),
        in_specs=in_specs,
        out_specs=acc_spec,
        out_shape=acc_shape,
    )


def _step_body(a3_ref, tm2_ref, dinv_ref, w_ref, accin_ref,
               t_ref, q3_ref, accout_ref, *, nq, alpha, beta):
    if nq == 2:
        a = jnp.concatenate([a3_ref[0], a3_ref[1]], axis=1)
    else:
        a = a3_ref[0]
    dinv = dinv_ref[...]
    t = alpha * dinv * a
    if beta != 0.0:
        t = t + beta * tm2_ref[...]
    t_ref[...] = t
    q = dinv * t
    for j in range(nq):
        q3_ref[j] = q[:, j * CH:(j + 1) * CH]
    accout_ref[...] = accin_ref[...] + jnp.dot(
        t, w_ref[...], preferred_element_type=jnp.float32)


@functools.cache
def _make_step(c, hdim, alpha, beta):
    nq = c // CH
    kern = functools.partial(_step_body, nq=nq, alpha=alpha, beta=beta)
    out_shape = [
        jax.ShapeDtypeStruct((N_PAD, c), jnp.float32),
        jax.ShapeDtypeStruct((nq, N_PAD, CH), jnp.float32),
        jax.ShapeDtypeStruct((N_PAD, hdim), jnp.float32),
    ]
    return pl.pallas_call(
        kern,
        grid=(GRID_TC,),
        in_specs=[
            pl.BlockSpec((nq, NB_TC, CH), lambda i: (0, i, 0)),
            pl.BlockSpec((NB_TC, c), lambda i: (i, 0)),
            pl.BlockSpec((NB_TC, 1), lambda i: (i, 0)),
            pl.BlockSpec((c, hdim), lambda i: (0, 0)),
            pl.BlockSpec((NB_TC, hdim), lambda i: (i, 0)),
        ],
        out_specs=[
            pl.BlockSpec((NB_TC, c), lambda i: (i, 0)),
            pl.BlockSpec((nq, NB_TC, CH), lambda i: (0, i, 0)),
            pl.BlockSpec((NB_TC, hdim), lambda i: (i, 0)),
        ],
        out_shape=out_shape,
    )


def _stats_body(acc_ref, s_ref):
    i = pl.program_id(0)

    @pl.when(i == 0)
    def _():
        s_ref[...] = jnp.zeros_like(s_ref)

    x = acc_ref[...]
    s0 = jnp.sum(x, axis=0, keepdims=True)
    s1 = jnp.sum(x * x, axis=0, keepdims=True)
    s_ref[...] += jnp.concatenate([s0, s1], axis=0)


@functools.cache
def _make_stats(hdim):
    return pl.pallas_call(
        _stats_body,
        grid=(GRID_TC,),
        in_specs=[pl.BlockSpec((NB_TC, hdim), lambda i: (i, 0))],
        out_specs=pl.BlockSpec((2, hdim), lambda i: (0, 0)),
        out_shape=jax.ShapeDtypeStruct((2, hdim), jnp.float32),
    )


def _norm_body(acc_ref, s_ref, g_ref, be_ref, dinv_ref, h_ref, q3_ref, *, ch):
    i = pl.program_id(0)
    inv_n = 1.0 / N
    mean = s_ref[0:1, :] * inv_n
    var = s_ref[1:2, :] * inv_n - mean * mean
    y = g_ref[...] * (acc_ref[...] - mean) * lax.rsqrt(var + 1e-5) + be_ref[...]
    y = jnp.maximum(y, 0.0)
    rows = i * NB_TC + lax.broadcasted_iota(jnp.int32, y.shape, 0)
    y = jnp.where(rows < N, y, 0.0)
    h_ref[...] = y
    q = y * dinv_ref[...]
    q3_ref[0] = q[:, :ch]
    q3_ref[1] = q[:, ch:]


@functools.cache
def _make_norm(hdim):
    ch = hdim // 2
    kern = functools.partial(_norm_body, ch=ch)
    out_shape = [
        jax.ShapeDtypeStruct((N_PAD, hdim), jnp.float32),
        jax.ShapeDtypeStruct((NCORE, N_PAD, ch), jnp.float32),
    ]
    return pl.pallas_call(
        kern,
        grid=(GRID_TC,),
        in_specs=[
            pl.BlockSpec((NB_TC, hdim), lambda i: (i, 0)),
            pl.BlockSpec((2, hdim), lambda i: (0, 0)),
            pl.BlockSpec((1, hdim), lambda i: (0, 0)),
            pl.BlockSpec((1, hdim), lambda i: (0, 0)),
            pl.BlockSpec((NB_TC, 1), lambda i: (i, 0)),
        ],
        out_specs=[
            pl.BlockSpec((NB_TC, hdim), lambda i: (i, 0)),
            pl.BlockSpec((NCORE, NB_TC, ch), lambda i: (0, i, 0)),
        ],
        out_shape=out_shape,
    )


def _bias_body(acc_ref, b_ref, out_ref):
    out_ref[...] = acc_ref[...] + b_ref[...]


@functools.cache
def _make_bias(hdim):
    return pl.pallas_call(
        _bias_body,
        grid=(GRID_TC,),
        in_specs=[
            pl.BlockSpec((NB_TC, hdim), lambda i: (i, 0)),
            pl.BlockSpec((1, hdim), lambda i: (0, 0)),
        ],
        out_specs=pl.BlockSpec((NB_TC, hdim), lambda i: (i, 0)),
        out_shape=jax.ShapeDtypeStruct((N_PAD, hdim), jnp.float32),
    )


def _cheb_layer(q3, h, dinv, ed, W):
    K, c, hdim = W.shape
    nq = c // CH
    segsum = _make_segsum(nq)
    blo, bhi = (ed["b2lo"], ed["b2hi"]) if nq == 2 else (ed["b1lo"], ed["b1hi"])
    if q3 is None:
        q3, acc = _make_init(c, hdim, True)(h, dinv, W[0])
    else:
        acc = _make_init(c, hdim, False)(h, dinv, W[0])
    t_km1, t_km2 = h, h
    for k in range(1, K):
        a3 = segsum(q3, ed["sd"], blo, bhi, ed["zeros"])
        alpha, beta = (-1.0, 0.0) if k == 1 else (-2.0, -1.0)
        t, q3, acc = _make_step(c, hdim, alpha, beta)(a3, t_km2, dinv, W[k], acc)
        t_km2, t_km1 = t_km1, t
    return acc


def kernel(x, edge_index, W1, b1, g1, be1, W2, b2, g2, be2,
           W3, b3, g3, be3, W4, b4):
    x = x.astype(jnp.float32)
    src = edge_index[0].astype(jnp.int32)
    dst = edge_index[1].astype(jnp.int32)
    pad_e = E_PAD - E
    srcp = jnp.concatenate([src, jnp.full((pad_e,), N, jnp.int32)])
    dstp = jnp.concatenate([dst, jnp.full((pad_e,), N, jnp.int32)])

    order = jnp.argsort(dstp)
    dst_s = dstp[order]
    src_s = srcp[order]

    sb_edges = SB_BLOCKS * EBLK
    nsb = -(-E_PAD // sb_edges) + 1
    extra = nsb * sb_edges - E_PAD
    src_sb = jnp.concatenate(
        [src_s, jnp.full((extra,), N, jnp.int32)]).reshape(-1, SB_BLOCKS, EBLK)
    dst_sb = jnp.concatenate(
        [dst_s, jnp.full((extra,), N, jnp.int32)]).reshape(-1, SB_BLOCKS, EBLK)
    sd = jnp.concatenate([src_sb, dst_sb], axis=1)

    def sb_bounds(step):
        nodes = jnp.arange(0, N_PAD + 1, step, dtype=jnp.int32)
        e = jnp.searchsorted(dst_s, nodes, side="left").astype(jnp.int32)
        lo = e[:-1] // sb_edges
        hi = -((-e[1:]) // sb_edges)
        return lo, hi

    lo64, hi64 = sb_bounds(N_PAD // 64)
    lo32, hi32 = sb_bounds(N_PAD // 32)
    ed = {
        "sd": sd,
        "b1lo": lo64.reshape(2, 2, 16),
        "b1hi": hi64.reshape(2, 2, 16),
        "b2lo": jnp.broadcast_to(lo32.reshape(2, 1, 16), (2, 2, 16)),
        "b2hi": jnp.broadcast_to(hi32.reshape(2, 1, 16), (2, 2, 16)),
        "zeros": jnp.zeros((N_PAD // NSUB + 8, CH), jnp.float32),
    }

    x_pad = jnp.pad(x, ((0, N_PAD - N), (0, 0)))

    a_deg = _make_deg()(srcp.reshape(E_PAD // EBLK, EBLK), ed["zeros"])
    dinv = _tc_prep(a_deg).reshape(N_PAD, 1)

    h = x_pad
    q3 = None
    for (W, g, be) in ((W1, g1, be1), (W2, g2, be2), (W3, g3, be3)):
        acc = _cheb_layer(q3, h, dinv, ed, W)
        s = _make_stats(W.shape[2])(acc)
        h, q3 = _make_norm(W.shape[2])(acc, s, g.reshape(1, -1),
                                       be.reshape(1, -1), dinv)
    acc = _cheb_layer(q3, h, dinv, ed, W4)
    out = _make_bias(W4.shape[2])(acc, b4.reshape(1, -1))
    return out[:N]


# speedup vs baseline: 1.0593x; 1.0495x over previous
import functools

import jax
import jax.numpy as jnp
from jax import lax
from jax.experimental import pallas as pl
from jax.experimental.pallas import tpu as pltpu
from jax.experimental.pallas import tpu_sc as plsc

N = 10000
N_PAD = 10240
E = 320000
NSUB = 16
NCORE = 2
EBLK = 128
NBLK = 160
E_PAD = NBLK * EBLK * NSUB
CH = 128
SB_BLOCKS = 4
PERF_PROBE_SKIP_SCATTER = False
ROWS_PER_SUB = N_PAD // NSUB
NB_TC = 1024
GRID_TC = N_PAD // NB_TC


def _extract_lane(vec, lane):
    ids = lax.broadcasted_iota(jnp.int32, (16,), 0)
    return jnp.sum(jnp.where(ids == lane, vec, 0))


@functools.cache
def _make_segsum(nq):
    mesh = plsc.VectorSubcoreMesh(core_axis_name="c", subcore_axis_name="s")
    acc_rows = N_PAD // 2 if nq == 2 else N_PAD // 4
    sub_rng = acc_rows // NSUB
    S = SB_BLOCKS

    def body(q3, sd2, blo, bhi, zeros_hbm, out3,
             blo_v, bhi_v, sdg, dstloc, rows_v, acc, sem_g, sem_s):
        cid = lax.axis_index("c")
        sid = lax.axis_index("s")
        table = q3.at[cid] if nq == 2 else q3.at[0]
        out = out3.at[cid] if nq == 2 else out3.at[0]
        pltpu.sync_copy(blo, blo_v)
        pltpu.sync_copy(bhi, bhi_v)

        def run_pass(p):
            if nq == 2:
                pbase = p * acc_rows
            else:
                pbase = (p * NCORE + cid) * acc_rows
            pltpu.sync_copy(zeros_hbm.at[pl.ds(0, sub_rng)],
                            acc.at[pl.ds(sid * sub_rng, sub_rng)])
            lo = _extract_lane(blo_v[p, cid], sid)
            hi = _extract_lane(bhi_v[p, cid], sid)
            nit = hi - lo
            vlo = sid * sub_rng

            def scatter_t(t):
                return pltpu.make_async_copy(
                    rows_v.at[t], acc.at[dstloc.at[t]], sem_s.at[t])

            def step(i, carry):
                pltpu.sync_copy(sd2.at[lo + i], sdg)
                for t in range(S):
                    if not PERF_PROBE_SKIP_SCATTER:
                        @pl.when(i > 0)
                        def _():
                            scatter_t(t).wait()

                    pltpu.async_copy(table.at[sdg.at[t]], rows_v.at[t],
                                     sem_g.at[t])
                for t in range(S):
                    for i8 in range(EBLK // 16):
                        d = sdg[S + t, pl.ds(16 * i8, 16)] - pbase
                        ok = jnp.logical_and(d >= vlo, d < vlo + sub_rng)
                        dstloc[t, pl.ds(16 * i8, 16)] = jnp.where(
                            ok, d, acc_rows)
                for t in range(S):
                    pltpu.make_async_copy(table.at[sdg.at[t]], rows_v.at[t],
                                          sem_g.at[t]).wait()
                    if not PERF_PROBE_SKIP_SCATTER:
                        pltpu.async_copy(rows_v.at[t], acc.at[dstloc.at[t]],
                                         sem_s.at[t], add=True)
                return carry

            lax.fori_loop(0, nit, step, 0)

            if not PERF_PROBE_SKIP_SCATTER:
                @pl.when(nit > 0)
                def _():
                    for t in range(S):
                        scatter_t(t).wait()

            pltpu.sync_copy(acc.at[pl.ds(sid * sub_rng, sub_rng)],
                            out.at[pl.ds(pbase + sid * sub_rng, sub_rng)])

        for p in range(2):
            run_pass(p)

    return pl.kernel(
        body,
        out_type=jax.ShapeDtypeStruct((nq, N_PAD, CH), jnp.float32),
        mesh=mesh,
        compiler_params=pltpu.CompilerParams(needs_layout_passes=False),
        scratch_types=[
            pltpu.VMEM((2, NCORE, 16), jnp.int32),
            pltpu.VMEM((2, NCORE, 16), jnp.int32),
            pltpu.VMEM((2 * S, EBLK), jnp.int32),
            pltpu.VMEM((S, EBLK), jnp.int32),
            pltpu.VMEM((S, EBLK, CH), jnp.float32),
            pltpu.VMEM_SHARED((acc_rows + 8, CH), jnp.float32),
            pltpu.SemaphoreType.DMA((S,)),
            pltpu.SemaphoreType.DMA((S,)),
        ],
    )


@functools.cache
def _make_deg():
    mesh = plsc.VectorSubcoreMesh(core_axis_name="c", subcore_axis_name="s")
    nrow = N_PAD // EBLK
    blocks_per_w = (E_PAD // EBLK) // (NCORE * NSUB)

    def body(src2, zeros_hbm, out3, srcg, iden, hloc, acc, sem):
        cid = lax.axis_index("c")
        sid = lax.axis_index("s")
        w = cid * NSUB + sid
        pltpu.sync_copy(zeros_hbm.at[pl.ds(0, nrow)], hloc)
        for i in range(nrow // 16):
            iden[0, pl.ds(16 * i, 16)] = (
                lax.broadcasted_iota(jnp.int32, (16,), 0) + 16 * i)
        @pl.when(sid == 0)
        def _():
            pltpu.sync_copy(zeros_hbm.at[pl.ds(0, nrow)], acc)

        plsc.subcore_barrier()
        ones = jnp.full((16,), 1.0, jnp.float32)

        def step(j, carry):
            pltpu.sync_copy(src2.at[j], srcg)
            for i in range(EBLK // 16):
                v = srcg[pl.ds(16 * i, 16)]
                plsc.addupdate_scatter(
                    hloc, [lax.shift_right_logical(v, 7),
                           lax.bitwise_and(v, 127)], ones)
            return carry

        lax.fori_loop(w * blocks_per_w, (w + 1) * blocks_per_w, step, 0)
        pltpu.sync_copy(hloc, acc.at[iden.at[0]], add=True)
        plsc.subcore_barrier()

        @pl.when(sid == 0)
        def _():
            pltpu.sync_copy(acc, out3.at[cid])

    return pl.kernel(
        body,
        out_type=jax.ShapeDtypeStruct((NCORE, nrow, EBLK), jnp.float32),
        mesh=mesh,
        compiler_params=pltpu.CompilerParams(needs_layout_passes=False),
        scratch_types=[
            pltpu.VMEM((EBLK,), jnp.int32),
            pltpu.VMEM((1, nrow), jnp.int32),
            pltpu.VMEM((nrow, EBLK), jnp.float32),
            pltpu.VMEM_SHARED((nrow, EBLK), jnp.float32),
            pltpu.SemaphoreType.DMA,
        ],
    )


def _prep_body(deg_ref, dinv_ref):
    d = deg_ref[0] + deg_ref[1]
    nrow = N_PAD // EBLK
    node = (lax.broadcasted_iota(jnp.int32, (nrow, EBLK), 0) * EBLK
            + lax.broadcasted_iota(jnp.int32, (nrow, EBLK), 1))
    ok = jnp.logical_and(d > 0.0, node < N)
    dinv_ref[...] = jnp.where(ok, lax.rsqrt(jnp.maximum(d, 1e-30)), 0.0)


def _tc_prep(degs):
    return pl.pallas_call(
        _prep_body,
        out_shape=jax.ShapeDtypeStruct((N_PAD // EBLK, EBLK), jnp.float32),
    )(degs)


def _init_body_q(h_ref, dinv_ref, w_ref, q3_ref, acc_ref, *, nq):
    h = h_ref[...]
    q = h * dinv_ref[...]
    for j in range(nq):
        q3_ref[j] = q[:, j * CH:(j + 1) * CH]
    acc_ref[...] = jnp.dot(h, w_ref[...], preferred_element_type=jnp.float32)


def _init_body_noq(h_ref, dinv_ref, w_ref, acc_ref):
    acc_ref[...] = jnp.dot(h_ref[...], w_ref[...],
                           preferred_element_type=jnp.float32)


@functools.cache
def _make_init(c, hdim, with_q):
    nq = c // CH
    in_specs = [
        pl.BlockSpec((NB_TC, c), lambda i: (i, 0)),
        pl.BlockSpec((NB_TC, 1), lambda i: (i, 0)),
        pl.BlockSpec((c, hdim), lambda i: (0, 0)),
    ]
    acc_spec = pl.BlockSpec((NB_TC, hdim), lambda i: (i, 0))
    acc_shape = jax.ShapeDtypeStruct((N_PAD, hdim), jnp.float32)
    if with_q:
        return pl.pallas_call(
            functools.partial(_init_body_q, nq=nq),
            grid=(GRID_TC,),
            in_specs=in_specs,
            out_specs=[
                pl.BlockSpec((nq, NB_TC, CH), lambda i: (0, i, 0)),
                acc_spec,
            ],
            out_shape=[
                jax.ShapeDtypeStruct((nq, N_PAD, CH), jnp.float32),
                acc_shape,
            ],
        )
    return pl.pallas_call(
        _init_body_noq,
        grid=(GRID_TC,),
        in_specs=in_specs,
        out_specs=acc_spec,
        out_shape=acc_shape,
    )


def _step_body(a3_ref, tm2_ref, dinv_ref, w_ref, accin_ref,
               t_ref, q3_ref, accout_ref, *, nq, alpha, beta):
    if nq == 2:
        a = jnp.concatenate([a3_ref[0], a3_ref[1]], axis=1)
    else:
        a = a3_ref[0]
    dinv = dinv_ref[...]
    t = alpha * dinv * a
    if beta != 0.0:
        t = t + beta * tm2_ref[...]
    t_ref[...] = t
    q = dinv * t
    for j in range(nq):
        q3_ref[j] = q[:, j * CH:(j + 1) * CH]
    accout_ref[...] = accin_ref[...] + jnp.dot(
        t, w_ref[...], preferred_element_type=jnp.float32)


@functools.cache
def _make_step(c, hdim, alpha, beta):
    nq = c // CH
    kern = functools.partial(_step_body, nq=nq, alpha=alpha, beta=beta)
    out_shape = [
        jax.ShapeDtypeStruct((N_PAD, c), jnp.float32),
        jax.ShapeDtypeStruct((nq, N_PAD, CH), jnp.float32),
        jax.ShapeDtypeStruct((N_PAD, hdim), jnp.float32),
    ]
    return pl.pallas_call(
        kern,
        grid=(GRID_TC,),
        in_specs=[
            pl.BlockSpec((nq, NB_TC, CH), lambda i: (0, i, 0)),
            pl.BlockSpec((NB_TC, c), lambda i: (i, 0)),
            pl.BlockSpec((NB_TC, 1), lambda i: (i, 0)),
            pl.BlockSpec((c, hdim), lambda i: (0, 0)),
            pl.BlockSpec((NB_TC, hdim), lambda i: (i, 0)),
        ],
        out_specs=[
            pl.BlockSpec((NB_TC, c), lambda i: (i, 0)),
            pl.BlockSpec((nq, NB_TC, CH), lambda i: (0, i, 0)),
            pl.BlockSpec((NB_TC, hdim), lambda i: (i, 0)),
        ],
        out_shape=out_shape,
    )


def _stats_body(acc_ref, s_ref):
    i = pl.program_id(0)

    @pl.when(i == 0)
    def _():
        s_ref[...] = jnp.zeros_like(s_ref)

    x = acc_ref[...]
    s0 = jnp.sum(x, axis=0, keepdims=True)
    s1 = jnp.sum(x * x, axis=0, keepdims=True)
    s_ref[...] += jnp.concatenate([s0, s1], axis=0)


@functools.cache
def _make_stats(hdim):
    return pl.pallas_call(
        _stats_body,
        grid=(GRID_TC,),
        in_specs=[pl.BlockSpec((NB_TC, hdim), lambda i: (i, 0))],
        out_specs=pl.BlockSpec((2, hdim), lambda i: (0, 0)),
        out_shape=jax.ShapeDtypeStruct((2, hdim), jnp.float32),
    )


def _norm_body(acc_ref, s_ref, g_ref, be_ref, dinv_ref, h_ref, q3_ref, *, ch):
    i = pl.program_id(0)
    inv_n = 1.0 / N
    mean = s_ref[0:1, :] * inv_n
    var = s_ref[1:2, :] * inv_n - mean * mean
    y = g_ref[...] * (acc_ref[...] - mean) * lax.rsqrt(var + 1e-5) + be_ref[...]
    y = jnp.maximum(y, 0.0)
    rows = i * NB_TC + lax.broadcasted_iota(jnp.int32, y.shape, 0)
    y = jnp.where(rows < N, y, 0.0)
    h_ref[...] = y
    q = y * dinv_ref[...]
    q3_ref[0] = q[:, :ch]
    q3_ref[1] = q[:, ch:]


@functools.cache
def _make_norm(hdim):
    ch = hdim // 2
    kern = functools.partial(_norm_body, ch=ch)
    out_shape = [
        jax.ShapeDtypeStruct((N_PAD, hdim), jnp.float32),
        jax.ShapeDtypeStruct((NCORE, N_PAD, ch), jnp.float32),
    ]
    return pl.pallas_call(
        kern,
        grid=(GRID_TC,),
        in_specs=[
            pl.BlockSpec((NB_TC, hdim), lambda i: (i, 0)),
            pl.BlockSpec((2, hdim), lambda i: (0, 0)),
            pl.BlockSpec((1, hdim), lambda i: (0, 0)),
            pl.BlockSpec((1, hdim), lambda i: (0, 0)),
            pl.BlockSpec((NB_TC, 1), lambda i: (i, 0)),
        ],
        out_specs=[
            pl.BlockSpec((NB_TC, hdim), lambda i: (i, 0)),
            pl.BlockSpec((NCORE, NB_TC, ch), lambda i: (0, i, 0)),
        ],
        out_shape=out_shape,
    )


def _bias_body(acc_ref, b_ref, out_ref):
    out_ref[...] = acc_ref[...] + b_ref[...]


@functools.cache
def _make_bias(hdim):
    return pl.pallas_call(
        _bias_body,
        grid=(GRID_TC,),
        in_specs=[
            pl.BlockSpec((NB_TC, hdim), lambda i: (i, 0)),
            pl.BlockSpec((1, hdim), lambda i: (0, 0)),
        ],
        out_specs=pl.BlockSpec((NB_TC, hdim), lambda i: (i, 0)),
        out_shape=jax.ShapeDtypeStruct((N_PAD, hdim), jnp.float32),
    )


def _cheb_layer(q3, h, dinv, ed, W):
    K, c, hdim = W.shape
    nq = c // CH
    segsum = _make_segsum(nq)
    blo, bhi = (ed["b2lo"], ed["b2hi"]) if nq == 2 else (ed["b1lo"], ed["b1hi"])
    if q3 is None:
        q3, acc = _make_init(c, hdim, True)(h, dinv, W[0])
    else:
        acc = _make_init(c, hdim, False)(h, dinv, W[0])
    t_km1, t_km2 = h, h
    for k in range(1, K):
        a3 = segsum(q3, ed["sd"], blo, bhi, ed["zeros"])
        alpha, beta = (-1.0, 0.0) if k == 1 else (-2.0, -1.0)
        t, q3, acc = _make_step(c, hdim, alpha, beta)(a3, t_km2, dinv, W[k], acc)
        t_km2, t_km1 = t_km1, t
    return acc


def kernel(x, edge_index, W1, b1, g1, be1, W2, b2, g2, be2,
           W3, b3, g3, be3, W4, b4):
    x = x.astype(jnp.float32)
    src = edge_index[0].astype(jnp.int32)
    dst = edge_index[1].astype(jnp.int32)
    pad_e = E_PAD - E
    srcp = jnp.concatenate([src, jnp.full((pad_e,), N, jnp.int32)])
    dstp = jnp.concatenate([dst, jnp.full((pad_e,), N, jnp.int32)])

    order = jnp.argsort(dstp)
    dst_s = dstp[order]
    src_s = srcp[order]

    sb_edges = SB_BLOCKS * EBLK
    nsb = -(-E_PAD // sb_edges) + 1
    extra = nsb * sb_edges - E_PAD
    src_sb = jnp.concatenate(
        [src_s, jnp.full((extra,), N, jnp.int32)]).reshape(-1, SB_BLOCKS, EBLK)
    dst_sb = jnp.concatenate(
        [dst_s, jnp.full((extra,), N, jnp.int32)]).reshape(-1, SB_BLOCKS, EBLK)
    sd = jnp.concatenate([src_sb, dst_sb], axis=1)

    def sb_bounds(step):
        nodes = jnp.arange(0, N_PAD + 1, step, dtype=jnp.int32)
        e = jnp.searchsorted(dst_s, nodes, side="left").astype(jnp.int32)
        lo = e[:-1] // sb_edges
        hi = -((-e[1:]) // sb_edges)
        return lo, hi

    lo64, hi64 = sb_bounds(N_PAD // 64)
    lo32, hi32 = sb_bounds(N_PAD // 32)
    ed = {
        "sd": sd,
        "b1lo": lo64.reshape(2, 2, 16),
        "b1hi": hi64.reshape(2, 2, 16),
        "b2lo": jnp.broadcast_to(lo32.reshape(2, 1, 16), (2, 2, 16)),
        "b2hi": jnp.broadcast_to(hi32.reshape(2, 1, 16), (2, 2, 16)),
        "zeros": jnp.zeros((N_PAD // NSUB + 8, CH), jnp.float32),
    }

    x_pad = jnp.pad(x, ((0, N_PAD - N), (0, 0)))

    a_deg = _make_deg()(srcp.reshape(E_PAD // EBLK, EBLK), ed["zeros"])
    dinv = _tc_prep(a_deg).reshape(N_PAD, 1)

    h = x_pad
    q3 = None
    for (W, g, be) in ((W1, g1, be1), (W2, g2, be2), (W3, g3, be3)):
        acc = _cheb_layer(q3, h, dinv, ed, W)
        s = _make_stats(W.shape[2])(acc)
        h, q3 = _make_norm(W.shape[2])(acc, s, g.reshape(1, -1),
                                       be.reshape(1, -1), dinv)
    acc = _cheb_layer(q3, h, dinv, ed, W4)
    out = _make_bias(W4.shape[2])(acc, b4.reshape(1, -1))
    return out[:N]


# speedup vs baseline: 1.5924x; 1.5032x over previous
import functools

import jax
import jax.numpy as jnp
from jax import lax
from jax.experimental import pallas as pl
from jax.experimental.pallas import tpu as pltpu
from jax.experimental.pallas import tpu_sc as plsc

N = 10000
N_PAD = 10240
E = 320000
NSUB = 16
NCORE = 2
EBLK = 128
NBLK = 160
E_PAD = NBLK * EBLK * NSUB
CH = 128
SB_BLOCKS = 3
WR = 3584
NSW = -(-N_PAD // WR)
SB_EDGES = SB_BLOCKS * EBLK
E_TOT = ((E // SB_EDGES) + 13) * SB_EDGES
NSB = E_TOT // SB_EDGES
ROWS_PER_SUB = N_PAD // NSUB
NB_TC = 1024
GRID_TC = N_PAD // NB_TC


def _extract_lane(vec, lane):
    ids = lax.broadcasted_iota(jnp.int32, (16,), 0)
    return jnp.sum(jnp.where(ids == lane, vec, 0))


@functools.cache
def _make_segsum(nq):
    mesh = plsc.VectorSubcoreMesh(core_axis_name="c", subcore_axis_name="s")
    acc_rows = N_PAD // 2 if nq == 2 else N_PAD // 4
    sub_rng = acc_rows // NSUB
    S = SB_BLOCKS

    def body(q3, sd2, blo, bhi, zeros_hbm, out3,
             blo_v, bhi_v, sdg, srcloc, dstloc, rows_v, qwin, acc,
             sem_g, sem_s):
        cid = lax.axis_index("c")
        sid = lax.axis_index("s")
        table = q3.at[cid] if nq == 2 else q3.at[0]
        out = out3.at[cid] if nq == 2 else out3.at[0]
        pltpu.sync_copy(blo, blo_v)
        pltpu.sync_copy(bhi, bhi_v)

        def scatter_t(t):
            return pltpu.make_async_copy(
                rows_v.at[t], acc.at[dstloc.at[t]], sem_s.at[t])

        def run_subpass(p, sw, dwbase):
            swbase = sw * WR
            wrc = min(WR, N_PAD - swbase)
            share = wrc // NSUB
            pltpu.sync_copy(table.at[pl.ds(swbase + sid * share, share)],
                            qwin.at[pl.ds(sid * share, share)])

            @pl.when(sid == 0)
            def _():
                pltpu.sync_copy(zeros_hbm.at[pl.ds(0, 8)],
                                qwin.at[pl.ds(WR, 8)])

            plsc.subcore_barrier()
            cix = (p * NCORE + cid) * NSW + sw
            clo = _extract_lane(blo_v[...], cix)
            span = _extract_lane(bhi_v[...], cix) - clo
            wlo = clo + lax.shift_right_logical(span * sid, 4)
            whi = clo + lax.shift_right_logical(span * (sid + 1), 4)
            nit = whi - wlo

            def step(i, carry):
                pltpu.sync_copy(sd2.at[wlo + i], sdg)
                for t in range(S):
                    for i8 in range(EBLK // 16):
                        sl = sdg[t, pl.ds(16 * i8, 16)] - swbase
                        ok = jnp.logical_and(sl >= 0, sl < wrc)
                        srcloc[t, pl.ds(16 * i8, 16)] = jnp.where(ok, sl, WR)
                for t in range(S):
                    @pl.when(i > 0)
                    def _():
                        scatter_t(t).wait()

                    pltpu.async_copy(qwin.at[srcloc.at[t]], rows_v.at[t],
                                     sem_g.at[t])
                for t in range(S):
                    for i8 in range(EBLK // 16):
                        dstloc[t, pl.ds(16 * i8, 16)] = (
                            sdg[4 + t, pl.ds(16 * i8, 16)] - dwbase)
                for t in range(S):
                    pltpu.make_async_copy(qwin.at[srcloc.at[t]],
                                          rows_v.at[t], sem_g.at[t]).wait()
                    pltpu.async_copy(rows_v.at[t], acc.at[dstloc.at[t]],
                                     sem_s.at[t], add=True)
                return carry

            lax.fori_loop(0, nit, step, 0)

            @pl.when(nit > 0)
            def _():
                for t in range(S):
                    scatter_t(t).wait()

            plsc.subcore_barrier()

        for p in range(2):
            if nq == 2:
                dwbase = p * acc_rows
            else:
                dwbase = (p * NCORE + cid) * acc_rows
            pltpu.sync_copy(zeros_hbm.at[pl.ds(0, sub_rng)],
                            acc.at[pl.ds(sid * sub_rng, sub_rng)])
            plsc.subcore_barrier()
            for sw in range(NSW):
                run_subpass(p, sw, dwbase)
            pltpu.sync_copy(acc.at[pl.ds(sid * sub_rng, sub_rng)],
                            out.at[pl.ds(dwbase + sid * sub_rng, sub_rng)])

    return pl.kernel(
        body,
        out_type=jax.ShapeDtypeStruct((nq, N_PAD, CH), jnp.float32),
        mesh=mesh,
        compiler_params=pltpu.CompilerParams(needs_layout_passes=False),
        scratch_types=[
            pltpu.VMEM((16,), jnp.int32),
            pltpu.VMEM((16,), jnp.int32),
            pltpu.VMEM((8, EBLK), jnp.int32),
            pltpu.VMEM((S, EBLK), jnp.int32),
            pltpu.VMEM((S, EBLK), jnp.int32),
            pltpu.VMEM((S, EBLK, CH), jnp.float32),
            pltpu.VMEM_SHARED((WR + 8, CH), jnp.float32),
            pltpu.VMEM_SHARED((acc_rows, CH), jnp.float32),
            pltpu.SemaphoreType.DMA((S,)),
            pltpu.SemaphoreType.DMA((S,)),
        ],
    )


@functools.cache
def _make_deg():
    mesh = plsc.VectorSubcoreMesh(core_axis_name="c", subcore_axis_name="s")
    nrow = N_PAD // EBLK
    blocks_per_w = (E_PAD // EBLK) // (NCORE * NSUB)

    def body(src2, zeros_hbm, out3, srcg, iden, hloc, acc, sem):
        cid = lax.axis_index("c")
        sid = lax.axis_index("s")
        w = cid * NSUB + sid
        pltpu.sync_copy(zeros_hbm.at[pl.ds(0, nrow)], hloc)
        for i in range(nrow // 16):
            iden[0, pl.ds(16 * i, 16)] = (
                lax.broadcasted_iota(jnp.int32, (16,), 0) + 16 * i)
        @pl.when(sid == 0)
        def _():
            pltpu.sync_copy(zeros_hbm.at[pl.ds(0, nrow)], acc)

        plsc.subcore_barrier()
        ones = jnp.full((16,), 1.0, jnp.float32)

        def step(j, carry):
            pltpu.sync_copy(src2.at[j], srcg)
            for i in range(EBLK // 16):
                v = srcg[pl.ds(16 * i, 16)]
                plsc.addupdate_scatter(
                    hloc, [lax.shift_right_logical(v, 7),
                           lax.bitwise_and(v, 127)], ones)
            return carry

        lax.fori_loop(w * blocks_per_w, (w + 1) * blocks_per_w, step, 0)
        pltpu.sync_copy(hloc, acc.at[iden.at[0]], add=True)
        plsc.subcore_barrier()

        @pl.when(sid == 0)
        def _():
            pltpu.sync_copy(acc, out3.at[cid])

    return pl.kernel(
        body,
        out_type=jax.ShapeDtypeStruct((NCORE, nrow, EBLK), jnp.float32),
        mesh=mesh,
        compiler_params=pltpu.CompilerParams(needs_layout_passes=False),
        scratch_types=[
            pltpu.VMEM((EBLK,), jnp.int32),
            pltpu.VMEM((1, nrow), jnp.int32),
            pltpu.VMEM((nrow, EBLK), jnp.float32),
            pltpu.VMEM_SHARED((nrow, EBLK), jnp.float32),
            pltpu.SemaphoreType.DMA,
        ],
    )


def _prep_body(deg_ref, dinv_ref):
    d = deg_ref[0] + deg_ref[1]
    nrow = N_PAD // EBLK
    node = (lax.broadcasted_iota(jnp.int32, (nrow, EBLK), 0) * EBLK
            + lax.broadcasted_iota(jnp.int32, (nrow, EBLK), 1))
    ok = jnp.logical_and(d > 0.0, node < N)
    dinv_ref[...] = jnp.where(ok, lax.rsqrt(jnp.maximum(d, 1e-30)), 0.0)


def _tc_prep(degs):
    return pl.pallas_call(
        _prep_body,
        out_shape=jax.ShapeDtypeStruct((N_PAD // EBLK, EBLK), jnp.float32),
    )(degs)


def _init_body_q(h_ref, dinv_ref, w_ref, q3_ref, acc_ref, *, nq):
    h = h_ref[...]
    q = h * dinv_ref[...]
    for j in range(nq):
        q3_ref[j] = q[:, j * CH:(j + 1) * CH]
    acc_ref[...] = jnp.dot(h, w_ref[...], preferred_element_type=jnp.float32)


def _init_body_noq(h_ref, dinv_ref, w_ref, acc_ref):
    acc_ref[...] = jnp.dot(h_ref[...], w_ref[...],
                           preferred_element_type=jnp.float32)


@functools.cache
def _make_init(c, hdim, with_q):
    nq = c // CH
    in_specs = [
        pl.BlockSpec((NB_TC, c), lambda i: (i, 0)),
        pl.BlockSpec((NB_TC, 1), lambda i: (i, 0)),
        pl.BlockSpec((c, hdim), lambda i: (0, 0)),
    ]
    acc_spec = pl.BlockSpec((NB_TC, hdim), lambda i: (i, 0))
    acc_shape = jax.ShapeDtypeStruct((N_PAD, hdim), jnp.float32)
    if with_q:
        return pl.pallas_call(
            functools.partial(_init_body_q, nq=nq),
            grid=(GRID_TC,),
            in_specs=in_specs,
            out_specs=[
                pl.BlockSpec((nq, NB_TC, CH), lambda i: (0, i, 0)),
                acc_spec,
            ],
            out_shape=[
                jax.ShapeDtypeStruct((nq, N_PAD, CH), jnp.float32),
                acc_shape,
            ],
        )
    return pl.pallas_call(
        _init_body_noq,
        grid=(GRID_TC,),
        in_specs=in_specs,
        out_specs=acc_spec,
        out_shape=acc_shape,
    )


def _step_body(a3_ref, tm2_ref, dinv_ref, w_ref, accin_ref,
               t_ref, q3_ref, accout_ref, *, nq, alpha, beta):
    if nq == 2:
        a = jnp.concatenate([a3_ref[0], a3_ref[1]], axis=1)
    else:
        a = a3_ref[0]
    dinv = dinv_ref[...]
    t = alpha * dinv * a
    if beta != 0.0:
        t = t + beta * tm2_ref[...]
    t_ref[...] = t
    q = dinv * t
    for j in range(nq):
        q3_ref[j] = q[:, j * CH:(j + 1) * CH]
    accout_ref[...] = accin_ref[...] + jnp.dot(
        t, w_ref[...], preferred_element_type=jnp.float32)


@functools.cache
def _make_step(c, hdim, alpha, beta):
    nq = c // CH
    kern = functools.partial(_step_body, nq=nq, alpha=alpha, beta=beta)
    out_shape = [
        jax.ShapeDtypeStruct((N_PAD, c), jnp.float32),
        jax.ShapeDtypeStruct((nq, N_PAD, CH), jnp.float32),
        jax.ShapeDtypeStruct((N_PAD, hdim), jnp.float32),
    ]
    return pl.pallas_call(
        kern,
        grid=(GRID_TC,),
        in_specs=[
            pl.BlockSpec((nq, NB_TC, CH), lambda i: (0, i, 0)),
            pl.BlockSpec((NB_TC, c), lambda i: (i, 0)),
            pl.BlockSpec((NB_TC, 1), lambda i: (i, 0)),
            pl.BlockSpec((c, hdim), lambda i: (0, 0)),
            pl.BlockSpec((NB_TC, hdim), lambda i: (i, 0)),
        ],
        out_specs=[
            pl.BlockSpec((NB_TC, c), lambda i: (i, 0)),
            pl.BlockSpec((nq, NB_TC, CH), lambda i: (0, i, 0)),
            pl.BlockSpec((NB_TC, hdim), lambda i: (i, 0)),
        ],
        out_shape=out_shape,
    )


def _stats_body(acc_ref, s_ref):
    i = pl.program_id(0)

    @pl.when(i == 0)
    def _():
        s_ref[...] = jnp.zeros_like(s_ref)

    x = acc_ref[...]
    s0 = jnp.sum(x, axis=0, keepdims=True)
    s1 = jnp.sum(x * x, axis=0, keepdims=True)
    s_ref[...] += jnp.concatenate([s0, s1], axis=0)


@functools.cache
def _make_stats(hdim):
    return pl.pallas_call(
        _stats_body,
        grid=(GRID_TC,),
        in_specs=[pl.BlockSpec((NB_TC, hdim), lambda i: (i, 0))],
        out_specs=pl.BlockSpec((2, hdim), lambda i: (0, 0)),
        out_shape=jax.ShapeDtypeStruct((2, hdim), jnp.float32),
    )


def _norm_body(acc_ref, s_ref, g_ref, be_ref, dinv_ref, h_ref, q3_ref, *, ch):
    i = pl.program_id(0)
    inv_n = 1.0 / N
    mean = s_ref[0:1, :] * inv_n
    var = s_ref[1:2, :] * inv_n - mean * mean
    y = g_ref[...] * (acc_ref[...] - mean) * lax.rsqrt(var + 1e-5) + be_ref[...]
    y = jnp.maximum(y, 0.0)
    rows = i * NB_TC + lax.broadcasted_iota(jnp.int32, y.shape, 0)
    y = jnp.where(rows < N, y, 0.0)
    h_ref[...] = y
    q = y * dinv_ref[...]
    q3_ref[0] = q[:, :ch]
    q3_ref[1] = q[:, ch:]


@functools.cache
def _make_norm(hdim):
    ch = hdim // 2
    kern = functools.partial(_norm_body, ch=ch)
    out_shape = [
        jax.ShapeDtypeStruct((N_PAD, hdim), jnp.float32),
        jax.ShapeDtypeStruct((NCORE, N_PAD, ch), jnp.float32),
    ]
    return pl.pallas_call(
        kern,
        grid=(GRID_TC,),
        in_specs=[
            pl.BlockSpec((NB_TC, hdim), lambda i: (i, 0)),
            pl.BlockSpec((2, hdim), lambda i: (0, 0)),
            pl.BlockSpec((1, hdim), lambda i: (0, 0)),
            pl.BlockSpec((1, hdim), lambda i: (0, 0)),
            pl.BlockSpec((NB_TC, 1), lambda i: (i, 0)),
        ],
        out_specs=[
            pl.BlockSpec((NB_TC, hdim), lambda i: (i, 0)),
            pl.BlockSpec((NCORE, NB_TC, ch), lambda i: (0, i, 0)),
        ],
        out_shape=out_shape,
    )


def _bias_body(acc_ref, b_ref, out_ref):
    out_ref[...] = acc_ref[...] + b_ref[...]


@functools.cache
def _make_bias(hdim):
    return pl.pallas_call(
        _bias_body,
        grid=(GRID_TC,),
        in_specs=[
            pl.BlockSpec((NB_TC, hdim), lambda i: (i, 0)),
            pl.BlockSpec((1, hdim), lambda i: (0, 0)),
        ],
        out_specs=pl.BlockSpec((NB_TC, hdim), lambda i: (i, 0)),
        out_shape=jax.ShapeDtypeStruct((N_PAD, hdim), jnp.float32),
    )


def _cheb_layer(q3, h, dinv, ed, W):
    K, c, hdim = W.shape
    nq = c // CH
    segsum = _make_segsum(nq)
    blo, bhi = (ed["b2lo"], ed["b2hi"]) if nq == 2 else (ed["b1lo"], ed["b1hi"])
    if q3 is None:
        q3, acc = _make_init(c, hdim, True)(h, dinv, W[0])
    else:
        acc = _make_init(c, hdim, False)(h, dinv, W[0])
    t_km1, t_km2 = h, h
    for k in range(1, K):
        a3 = segsum(q3, ed["sd"], blo, bhi, ed["zeros"])
        alpha, beta = (-1.0, 0.0) if k == 1 else (-2.0, -1.0)
        t, q3, acc = _make_step(c, hdim, alpha, beta)(a3, t_km2, dinv, W[k], acc)
        t_km2, t_km1 = t_km1, t
    return acc


def kernel(x, edge_index, W1, b1, g1, be1, W2, b2, g2, be2,
           W3, b3, g3, be3, W4, b4):
    x = x.astype(jnp.float32)
    src = edge_index[0].astype(jnp.int32)
    dst = edge_index[1].astype(jnp.int32)
    pad_e = E_PAD - E
    srcp = jnp.concatenate([src, jnp.full((pad_e,), N, jnp.int32)])
    dstp = jnp.concatenate([dst, jnp.full((pad_e,), N, jnp.int32)])

    half = dst // (N_PAD // 2)
    qpar = (dst // (N_PAD // 4)) % 2
    swin = src // WR
    bucket = half * (2 * NSW) + swin * 2 + qpar
    counts = jnp.bincount(bucket, length=12)
    order = jnp.argsort(bucket)
    b_sorted = bucket[order]
    ustart = jnp.concatenate(
        [jnp.zeros((1,), jnp.int32), jnp.cumsum(counts)[:-1].astype(jnp.int32)])
    psize = ((counts + SB_EDGES - 1) // SB_EDGES) * SB_EDGES
    pstart = jnp.concatenate(
        [jnp.zeros((1,), jnp.int32), jnp.cumsum(psize)[:-1].astype(jnp.int32)])
    pend = (pstart + psize).astype(jnp.int32)
    pos = pstart[b_sorted] + jnp.arange(E, dtype=jnp.int32) - ustart[b_sorted]
    bidx = jnp.arange(12, dtype=jnp.int32)
    qbase = ((bidx // (2 * NSW)) * 2 + (bidx % 2)) * (N_PAD // 4)
    slot_bucket = jnp.minimum(
        jnp.searchsorted(pend, jnp.arange(E_TOT, dtype=jnp.int32),
                         side="right"), 11)
    srcp_f = jnp.full((E_TOT,), jnp.int32(1 << 30)).at[pos].set(src[order])
    dstp_f = qbase[slot_bucket].astype(jnp.int32).at[pos].set(dst[order])
    src_sb = srcp_f.reshape(NSB, SB_BLOCKS, EBLK)
    dst_sb = dstp_f.reshape(NSB, SB_BLOCKS, EBLK)
    pad_row = jnp.zeros((NSB, 1, EBLK), jnp.int32)
    sd = jnp.concatenate([src_sb, pad_row, dst_sb, pad_row], axis=1)

    def cell_bounds(nq):
        lo, hi = [], []
        for p in range(2):
            for c in range(2):
                for sw in range(NSW):
                    if nq == 2:
                        b0 = p * (2 * NSW) + sw * 2
                        lo.append(pstart[b0] // SB_EDGES)
                        hi.append(pend[b0 + 1] // SB_EDGES)
                    else:
                        q = p * 2 + c
                        b = (q // 2) * (2 * NSW) + sw * 2 + (q % 2)
                        lo.append(pstart[b] // SB_EDGES)
                        hi.append(pend[b] // SB_EDGES)
        pad = [jnp.int32(0)] * (16 - len(lo))
        return (jnp.stack(lo + pad).astype(jnp.int32),
                jnp.stack(hi + pad).astype(jnp.int32))

    b1lo, b1hi = cell_bounds(1)
    b2lo, b2hi = cell_bounds(2)
    ed = {
        "sd": sd,
        "b1lo": b1lo, "b1hi": b1hi, "b2lo": b2lo, "b2hi": b2hi,
        "zeros": jnp.zeros((N_PAD // NSUB + 8, CH), jnp.float32),
    }

    x_pad = jnp.pad(x, ((0, N_PAD - N), (0, 0)))

    a_deg = _make_deg()(srcp.reshape(E_PAD // EBLK, EBLK), ed["zeros"])
    dinv = _tc_prep(a_deg).reshape(N_PAD, 1)

    h = x_pad
    q3 = None
    for (W, g, be) in ((W1, g1, be1), (W2, g2, be2), (W3, g3, be3)):
        acc = _cheb_layer(q3, h, dinv, ed, W)
        s = _make_stats(W.shape[2])(acc)
        h, q3 = _make_norm(W.shape[2])(acc, s, g.reshape(1, -1),
                                       be.reshape(1, -1), dinv)
    acc = _cheb_layer(q3, h, dinv, ed, W4)
    out = _make_bias(W4.shape[2])(acc, b4.reshape(1, -1))
    return out[:N]
